# Initial kernel scaffold; baseline (speedup 1.0000x reference)
import jax
import jax.numpy as jnp
from jax.experimental import pallas as pl


def kernel(embedding, memory_bank):
    raise NotImplementedError("write your pallas kernel here")


# speedup vs baseline: 16.8273x; 16.8273x over previous
import numpy as np
import jax
import jax.numpy as jnp
from jax.experimental import pallas as pl

_pc = pl.pallas_call

_BANK = 100000
_CHUNK = 2048
_NCHUNK = 49
_PAD = _CHUNK * _NCHUNK
_NQ = 1024
_DIM = 32
_K = 9


def _gauss_upsample_matrix():
    ks = 33
    sigma = 4.0
    xs = np.arange(ks, dtype=np.float64) - (ks - 1) * 0.5
    g = np.exp(-(xs ** 2) / (2.0 * sigma * sigma))
    g = g / g.sum()
    w = np.zeros((224, 32), dtype=np.float64)
    for i in range(224):
        for t in range(ks):
            r = i + t - 16
            if r < 0:
                r = -r
            elif r > 223:
                r = 446 - r
            w[i, r // 7] += g[t]
    return w.astype(np.float32)


_W_NP = _gauss_upsample_matrix()


def _min_d2_kernel(em2_ref, et_ref, m_ref, o_ref):
    i = pl.program_id(0)
    m = m_ref[...]
    b2 = jnp.sum(m * m, axis=1, keepdims=True)
    row = jax.lax.broadcasted_iota(jnp.int32, (_CHUNK, 1), 0) + i * _CHUNK
    b2 = jnp.where(row < _BANK, b2, jnp.inf)
    g = jax.lax.dot_general(m, em2_ref[...], (((1,), (1,)), ((), ())),
                            preferred_element_type=jnp.float32)
    d2 = b2 + g
    cmin = jnp.min(d2, axis=0, keepdims=True)
    acc = jnp.where(i == 0, cmin, jnp.minimum(o_ref[...], cmin))
    et = et_ref[...]
    a2 = jnp.sum(et * et, axis=0, keepdims=True)
    scores = jnp.sqrt(jnp.maximum(acc + a2, 0.0))
    o_ref[...] = jnp.where(i == _NCHUNK - 1, scores, acc)


def _row_d2_kernel(em2_ref, mt_ref, o_ref):
    mt = mt_ref[...]
    b2 = jnp.sum(mt * mt, axis=0, keepdims=True)
    g = jax.lax.dot_general(em2_ref[...], mt, (((1,), (0,)), ((), ())),
                            preferred_element_type=jnp.float32)
    o_ref[...] = b2 + g


def _top9_kernel(d_ref, o_ref):
    d = d_ref[...]
    n = (jax.lax.broadcasted_iota(jnp.int32, d.shape, 0) * 128
         + jax.lax.broadcasted_iota(jnp.int32, d.shape, 1))
    d = jnp.where(n < _BANK, d, jnp.inf)
    lane = jax.lax.broadcasted_iota(jnp.int32, (1, 128), 1)
    vals = jnp.zeros((1, 128), jnp.float32)
    for j in range(_K):
        v = jnp.min(d)
        idx = jnp.min(jnp.where(d == v, n, jnp.int32(2147483647)))
        vals = jnp.where(lane == j, v, vals)
        d = jnp.where(n == idx, jnp.inf, d)
    o_ref[...] = vals


def _blur_kernel(w_ref, wt_ref, x_ref, o_ref):
    t = jax.lax.dot_general(w_ref[...], x_ref[...], (((1,), (0,)), ((), ())),
                            preferred_element_type=jnp.float32)
    o_ref[...] = jax.lax.dot_general(t, wt_ref[...], (((1,), (0,)), ((), ())),
                                     preferred_element_type=jnp.float32)


def kernel(embedding, memory_bank):
    e = embedding.astype(jnp.float32)
    m = jnp.pad(memory_bank.astype(jnp.float32), ((0, _PAD - _BANK), (0, 0)))
    em2 = -2.0 * e
    et = e.T
    mt = m.T

    minref = _pc(
        _min_d2_kernel,
        grid=(_NCHUNK,),
        in_specs=[
            pl.BlockSpec((_NQ, _DIM), lambda i: (0, 0)),
            pl.BlockSpec((_DIM, _NQ), lambda i: (0, 0)),
            pl.BlockSpec((_CHUNK, _DIM), lambda i: (i, 0)),
        ],
        out_specs=pl.BlockSpec((1, _NQ), lambda i: (0, 0)),
        out_shape=jax.ShapeDtypeStruct((1, _NQ), jnp.float32),
    )(em2, et, m)
    scores = minref[0]

    w = jnp.asarray(_W_NP)
    m32 = scores.reshape(32, 32)
    amap = _pc(
        _blur_kernel,
        out_shape=jax.ShapeDtypeStruct((224, 224), jnp.float32),
    )(w, w.T, m32)
    amap = amap.reshape(1, 1, 224, 224)

    max_idx = jnp.argmax(scores)
    e_row = jax.lax.dynamic_slice(e, (max_idx, jnp.int32(0)), (1, _DIM))
    em2_row = -2.0 * e_row
    d2row = _pc(
        _row_d2_kernel,
        grid=(_NCHUNK,),
        in_specs=[
            pl.BlockSpec((1, _DIM), lambda i: (0, 0)),
            pl.BlockSpec((_DIM, _CHUNK), lambda i: (0, i)),
        ],
        out_specs=pl.BlockSpec((1, _CHUNK), lambda i: (0, i)),
        out_shape=jax.ShapeDtypeStruct((1, _PAD), jnp.float32),
    )(em2_row, mt)
    top = _pc(
        _top9_kernel,
        out_shape=jax.ShapeDtypeStruct((1, 128), jnp.float32),
    )(d2row.reshape(784, 128))
    a2row = jnp.sum(e_row * e_row)
    conf = jnp.sqrt(jnp.maximum(top[0, :_K] + a2row, 0.0))
    ec = jnp.exp(conf)
    weights = 1.0 - jnp.max(ec) / jnp.sum(ec)
    anomaly_score = weights * jnp.max(scores)
    return (amap, anomaly_score)


# speedup vs baseline: 19.4405x; 1.1553x over previous
import numpy as np
import jax
import jax.numpy as jnp
from jax.experimental import pallas as pl

_pc = pl.pallas_call

_BANK = 100000
_CHUNK = 2000
_NCHUNK = 50
_NQ = 1024
_DIM = 32
_K = 9
_CHUNK2 = 2048
_NCHUNK2 = 49
_PAD2 = _CHUNK2 * _NCHUNK2
_TR = 784
_TL = 128


def _gauss_upsample_matrix():
    ks = 33
    sigma = 4.0
    xs = np.arange(ks, dtype=np.float64) - (ks - 1) * 0.5
    g = np.exp(-(xs ** 2) / (2.0 * sigma * sigma))
    g = g / g.sum()
    w = np.zeros((224, 32), dtype=np.float64)
    for i in range(224):
        for t in range(ks):
            r = i + t - 16
            if r < 0:
                r = -r
            elif r > 223:
                r = 446 - r
            w[i, r // 7] += g[t]
    return w.astype(np.float32)


_W_NP = _gauss_upsample_matrix()


def _min_d2_kernel(em2_ref, et_ref, m_ref, o_ref):
    i = pl.program_id(0)
    m = m_ref[...]
    b2 = jnp.sum(m * m, axis=1, keepdims=True)
    g = jax.lax.dot_general(m, em2_ref[...], (((1,), (1,)), ((), ())),
                            preferred_element_type=jnp.float32)
    d2 = b2 + g
    cmin = jnp.min(d2, axis=0, keepdims=True)
    acc = jnp.where(i == 0, cmin, jnp.minimum(o_ref[...], cmin))
    et = et_ref[...]
    a2 = jnp.sum(et * et, axis=0, keepdims=True)
    scores = jnp.sqrt(jnp.maximum(acc + a2, 0.0))
    o_ref[...] = jnp.where(i == _NCHUNK - 1, scores, acc)


def _row_d2_kernel(ea_ref, m_ref, o_ref):
    m = m_ref[...]
    cat = jnp.concatenate([m, m * m], axis=1)
    o_ref[...] = jax.lax.dot_general(
        ea_ref[...], cat, (((1,), (1,)), ((), ())),
        preferred_element_type=jnp.float32)


def _top9_kernel(d_ref, o_ref):
    d = d_ref[...]
    n = (jax.lax.broadcasted_iota(jnp.int32, d.shape, 0) * _TL
         + jax.lax.broadcasted_iota(jnp.int32, d.shape, 1))
    d = jnp.where(n < _BANK, d, jnp.inf)
    lane = jax.lax.broadcasted_iota(jnp.int32, (1, 128), 1)
    vals = jnp.zeros((1, 128), jnp.float32)
    for j in range(_K):
        v = jnp.min(d)
        idx = jnp.min(jnp.where(d == v, n, jnp.int32(2147483647)))
        vals = jnp.where(lane == j, v, vals)
        d = jnp.where(n == idx, jnp.inf, d)
    o_ref[...] = vals


def _blur_kernel(w_ref, wt_ref, x_ref, o_ref):
    t = jax.lax.dot_general(w_ref[...], x_ref[...], (((1,), (0,)), ((), ())),
                            preferred_element_type=jnp.float32)
    o_ref[...] = jax.lax.dot_general(t, wt_ref[...], (((1,), (0,)), ((), ())),
                                     preferred_element_type=jnp.float32)


def kernel(embedding, memory_bank):
    e = embedding.astype(jnp.float32)
    m = memory_bank.astype(jnp.float32)
    em2 = -2.0 * e
    et = e.T

    minref = _pc(
        _min_d2_kernel,
        grid=(_NCHUNK,),
        in_specs=[
            pl.BlockSpec((_NQ, _DIM), lambda i: (0, 0)),
            pl.BlockSpec((_DIM, _NQ), lambda i: (0, 0)),
            pl.BlockSpec((_CHUNK, _DIM), lambda i: (i, 0)),
        ],
        out_specs=pl.BlockSpec((1, _NQ), lambda i: (0, 0)),
        out_shape=jax.ShapeDtypeStruct((1, _NQ), jnp.float32),
    )(em2, et, m)
    scores = minref[0]

    w = jnp.asarray(_W_NP)
    amap = _pc(
        _blur_kernel,
        out_shape=jax.ShapeDtypeStruct((224, 224), jnp.float32),
    )(w, w.T, scores.reshape(32, 32))
    amap = amap.reshape(1, 1, 224, 224)

    max_idx = jnp.argmax(scores)
    e_row = jax.lax.dynamic_slice(e, (max_idx, jnp.int32(0)), (1, _DIM))
    ea_row = jnp.concatenate([-2.0 * e_row, jnp.ones((1, _DIM), jnp.float32)],
                             axis=1)
    d2row = _pc(
        _row_d2_kernel,
        grid=(_NCHUNK2,),
        in_specs=[
            pl.BlockSpec((1, 2 * _DIM), lambda i: (0, 0)),
            pl.BlockSpec((_CHUNK2, _DIM), lambda i: (i, 0)),
        ],
        out_specs=pl.BlockSpec((1, _CHUNK2), lambda i: (0, i)),
        out_shape=jax.ShapeDtypeStruct((1, _PAD2), jnp.float32),
    )(ea_row, m)
    top = _pc(
        _top9_kernel,
        out_shape=jax.ShapeDtypeStruct((1, 128), jnp.float32),
    )(d2row.reshape(_TR, _TL))
    a2row = jnp.sum(e_row * e_row)
    conf = jnp.sqrt(jnp.maximum(top[0, :_K] + a2row, 0.0))
    ec = jnp.exp(conf)
    weights = 1.0 - jnp.max(ec) / jnp.sum(ec)
    anomaly_score = weights * jnp.max(scores)
    return (amap, anomaly_score)
